# baseline (speedup 1.0000x reference)
import functools

import jax
import jax.numpy as jnp
from jax import lax
from jax.experimental import pallas as pl
from jax.experimental.pallas import tpu as pltpu
from jax.experimental.pallas import tpu_sc as plsc

_E = 8
_K = 2
_T = 2048
_D = 1024
_F = 2048
_B = 256
_P = _T * _K
_NPAD = _P + _E * _B
_NBLK = _NPAD // _B
_NW = 32
_RPW = _NPAD // _NW
_TPW = _T // _NW


def _route_body(topk_ref, w_ref, lmap_ref, posall_ref, bexp_ref):
    f32 = jnp.float32
    i32 = jnp.int32
    ii = lax.broadcasted_iota(i32, (128, 128), 0)
    jj = lax.broadcasted_iota(i32, (128, 128), 1)
    U128 = (ii < jj).astype(f32)
    ones128 = jnp.ones((128, 128), f32)
    i16 = lax.broadcasted_iota(i32, (16, 16), 0)
    j16 = lax.broadcasted_iota(i32, (16, 16), 1)
    L16 = (j16 < i16).astype(f32)

    def mm(a, b):
        return lax.dot_general(a, b, (((1,), (0,)), ((), ())),
                               preferred_element_type=f32)

    loc = []
    for k in range(_K):
        tk = topk_ref[k]
        lk = jnp.zeros_like(tk)
        for j in range(_E):
            lk = jnp.where(tk == j, lmap_ref[j], lk)
        loc.append(lk)

    oh = [[None] * _E for _ in range(_K)]
    rank = [[None] * _E for _ in range(_K)]
    c_col = [[None] * _E for _ in range(_K)]
    for k in range(_K):
        for e in range(_E):
            o = (loc[k] == e).astype(f32)
            r = mm(o, U128) + mm(L16, mm(o, ones128))
            oh[k][e] = o
            rank[k][e] = r
            c_col[k][e] = jnp.sum(o)

    start = []
    blkstart = []
    pad = []
    P_cum = []
    s = jnp.float32(0.0)
    pcum = jnp.float32(0.0)
    for e in range(_E):
        ce = c_col[0][e] + c_col[1][e]
        used_rows = jnp.floor((ce + (_B - 1)) * (1.0 / _B)) * _B
        start.append(s)
        blkstart.append(s * (1.0 / _B))
        pad.append(used_rows - ce)
        P_cum.append(pcum)
        s = s + used_rows
        pcum = pcum + (used_rows - ce)
    total_used = s
    P_tail = pcum

    for k in range(_K):
        p = jnp.zeros((16, 128), f32)
        for e in range(_E):
            base = start[e] + (c_col[0][e] if k == 1 else 0.0)
            p = p + oh[k][e] * (base + rank[k][e])
        posall_ref[16 * k:16 * (k + 1)] = p.astype(jnp.int32)

    jmat = (lax.broadcasted_iota(i32, (16, 128), 0) * 128
            + lax.broadcasted_iota(i32, (16, 128), 1)).astype(f32)
    pp = jnp.zeros((16, 128), f32)
    for e in range(_E):
        lo, hi = P_cum[e], P_cum[e] + pad[e]
        m = jnp.logical_and(jmat >= lo, jmat < hi).astype(f32)
        pp = pp + m * (start[e] + (c_col[0][e] + c_col[1][e]) + (jmat - lo))
    mtail = (jmat >= P_tail).astype(f32)
    pp = pp + mtail * (total_used + (jmat - P_tail))
    posall_ref[32:48] = pp.astype(jnp.int32)

    bi = lax.broadcasted_iota(i32, (8, 128), 1).astype(f32)
    be = jnp.zeros((8, 128), f32)
    for e in range(_E):
        lo = blkstart[e]
        hi = blkstart[e + 1] if e + 1 < _E else total_used * (1.0 / _B)
        m = jnp.logical_and(bi >= lo, bi < hi).astype(f32)
        be = be + m * e
    bexp_ref[...] = be.astype(jnp.int32)


def _route(topk_km, w_km, lmap):
    return pl.pallas_call(
        _route_body,
        in_specs=[
            pl.BlockSpec((_K, 16, 128), lambda: (0, 0, 0)),
            pl.BlockSpec((_K, 16, 128), lambda: (0, 0, 0)),
            pl.BlockSpec(memory_space=pltpu.SMEM),
        ],
        out_specs=[
            pl.BlockSpec((48, 128), lambda: (0, 0)),
            pl.BlockSpec((8, 128), lambda: (0, 0)),
        ],
        out_shape=[
            jax.ShapeDtypeStruct((48, 128), jnp.int32),
            jax.ShapeDtypeStruct((8, 128), jnp.int32),
        ],
    )(topk_km, w_km, lmap)


def _make_scatter():
    mesh = plsc.VectorSubcoreMesh(core_axis_name="c", subcore_axis_name="s")

    @functools.partial(
        pl.kernel, mesh=mesh,
        out_type=[
            jax.ShapeDtypeStruct((_NPAD,), jnp.int32),
            jax.ShapeDtypeStruct((_NPAD,), jnp.float32),
        ],
        scratch_types=[
            pltpu.VMEM((3, 64), jnp.int32),
            pltpu.VMEM((64,), jnp.int32),
            pltpu.VMEM((64,), jnp.float32),
            pltpu.SemaphoreType.DMA,
        ],
    )
    def k(pos_hbm, tok_hbm, w_hbm, stok_hbm, roww_hbm, idx_v, tok_v, w_v, sem):
        wid = lax.axis_index("s") * 2 + lax.axis_index("c")
        base = wid * _RPW
        for c in range(3):
            off = base + c * 64
            pltpu.sync_copy(pos_hbm.at[pl.ds(off, 64)], idx_v.at[c])
            pltpu.sync_copy(tok_hbm.at[pl.ds(off, 64)], tok_v)
            pltpu.async_copy(tok_v, stok_hbm.at[idx_v.at[c]], sem).wait()
            pltpu.sync_copy(w_hbm.at[pl.ds(off, 64)], w_v)
            pltpu.async_copy(w_v, roww_hbm.at[idx_v.at[c]], sem).wait()

    return k


def _make_gather():
    mesh = plsc.VectorSubcoreMesh(core_axis_name="c", subcore_axis_name="s")

    @functools.partial(
        pl.kernel, mesh=mesh,
        out_type=jax.ShapeDtypeStruct((_NPAD, _D), jnp.float32),
        scratch_types=[
            pltpu.VMEM((64,), jnp.int32),
            pltpu.VMEM((64, _D), jnp.float32),
            pltpu.SemaphoreType.DMA,
        ],
    )
    def k(stok_hbm, x_hbm, xs_hbm, idx_v, rows_v, sem):
        wid = lax.axis_index("s") * 2 + lax.axis_index("c")
        base = wid * _RPW
        for c in range(3):
            off = base + c * 64
            pltpu.sync_copy(stok_hbm.at[pl.ds(off, 64)], idx_v)
            pltpu.async_copy(x_hbm.at[idx_v], rows_v, sem).wait()
            pltpu.sync_copy(rows_v, xs_hbm.at[pl.ds(off, 64)])

    return k


def _ffn_body(bexp_ref, xs_ref, w1_ref, w2_ref, roww_ref, o_ref):
    h = jnp.maximum(
        lax.dot_general(xs_ref[...], w1_ref[...], (((1,), (0,)), ((), ())),
                        preferred_element_type=jnp.float32), 0.0)
    o = lax.dot_general(h, w2_ref[...], (((1,), (0,)), ((), ())),
                        preferred_element_type=jnp.float32)
    o_ref[...] = o * roww_ref[0][:, None]


def _ffn(bexp, xs, W1, W2, roww3):
    grid_spec = pltpu.PrefetchScalarGridSpec(
        num_scalar_prefetch=1,
        grid=(_NBLK,),
        in_specs=[
            pl.BlockSpec((_B, _D), lambda b, be: (b, 0)),
            pl.BlockSpec((None, _D, _F), lambda b, be: (be[b], 0, 0)),
            pl.BlockSpec((None, _F, _D), lambda b, be: (be[b], 0, 0)),
            pl.BlockSpec((None, 1, _B), lambda b, be: (b, 0, 0)),
        ],
        out_specs=pl.BlockSpec((_B, _D), lambda b, be: (b, 0)),
    )
    return pl.pallas_call(
        _ffn_body,
        grid_spec=grid_spec,
        out_shape=jax.ShapeDtypeStruct((_NPAD, _D), jnp.float32),
    )(bexp, xs, W1, W2, roww3)


def _make_combine():
    mesh = plsc.VectorSubcoreMesh(core_axis_name="c", subcore_axis_name="s")

    @functools.partial(
        pl.kernel, mesh=mesh,
        out_type=jax.ShapeDtypeStruct((_T, _D), jnp.float32),
        scratch_types=[
            pltpu.VMEM((32,), jnp.int32),
            pltpu.VMEM((32, _D), jnp.float32),
            pltpu.VMEM((32, _D), jnp.float32),
            pltpu.SemaphoreType.DMA,
        ],
    )
    def k(o_hbm, pos0_hbm, pos1_hbm, y_hbm, idx_v, g0_v, g1_v, sem):
        wid = lax.axis_index("s") * 2 + lax.axis_index("c")
        base = wid * _TPW
        for c in range(2):
            off = base + c * 32
            pltpu.sync_copy(pos0_hbm.at[pl.ds(off, 32)], idx_v)
            pltpu.async_copy(o_hbm.at[idx_v], g0_v, sem).wait()
            pltpu.sync_copy(pos1_hbm.at[pl.ds(off, 32)], idx_v)
            pltpu.async_copy(o_hbm.at[idx_v], g1_v, sem).wait()
            def add_row(r, carry):
                for v in range(_D // 16):
                    sl = pl.ds(v * 16, 16)
                    g0_v[r, sl] = g0_v[r, sl] + g1_v[r, sl]
                return carry

            lax.fori_loop(0, 32, add_row, 0)
            pltpu.sync_copy(g0_v, y_hbm.at[pl.ds(off, 32)])

    return k


def kernel(x, topk_indices, topk_weights, W1, W2, device_indices_map,
           local_expert_indices_map):
    topk_km = topk_indices.T.reshape(_K, 16, 128)
    w_km = topk_weights.T.reshape(_K, 16, 128)
    posall, bexp = _route(topk_km, w_km, local_expert_indices_map)
    pos_flat = posall.reshape(-1)
    bexp_vec = bexp.reshape(-1)[:_NBLK]

    tok_all = jnp.concatenate([
        jnp.arange(_T, dtype=jnp.int32),
        jnp.arange(_T, dtype=jnp.int32),
        jnp.zeros((_NPAD - _P,), jnp.int32),
    ])
    w_all = jnp.concatenate([
        topk_weights.T.reshape(-1),
        jnp.zeros((_NPAD - _P,), jnp.float32),
    ])

    stok, roww = _make_scatter()(pos_flat, tok_all, w_all)
    xs = _make_gather()(stok, x)
    o = _ffn(bexp_vec, xs, W1, W2, roww.reshape(_NBLK, 1, _B))
    y = _make_combine()(o, pos_flat[:_T], pos_flat[_T:_P])
    return y


# speedup vs baseline: 1.2001x; 1.2001x over previous
import functools

import jax
import jax.numpy as jnp
from jax import lax
from jax.experimental import pallas as pl
from jax.experimental.pallas import tpu as pltpu
from jax.experimental.pallas import tpu_sc as plsc

_E = 8
_K = 2
_T = 2048
_D = 1024
_F = 2048
_B = 256
_P = _T * _K
_NPAD = _P + _E * _B
_NBLK = _NPAD // _B
_NW = 32
_RPW = _NPAD // _NW
_GC = 48
_TPW = _T // _NW


def _route_body(topk_ref, w_ref, lmap_ref, posall_ref, bexp_ref):
    f32 = jnp.float32
    i32 = jnp.int32
    ii = lax.broadcasted_iota(i32, (128, 128), 0)
    jj = lax.broadcasted_iota(i32, (128, 128), 1)
    U128 = (ii < jj).astype(f32)
    ones128 = jnp.ones((128, 128), f32)
    i16 = lax.broadcasted_iota(i32, (16, 16), 0)
    j16 = lax.broadcasted_iota(i32, (16, 16), 1)
    L16 = (j16 < i16).astype(f32)

    def mm(a, b):
        return lax.dot_general(a, b, (((1,), (0,)), ((), ())),
                               preferred_element_type=f32)

    loc = []
    for k in range(_K):
        tk = topk_ref[k]
        lk = jnp.zeros_like(tk)
        for j in range(_E):
            lk = jnp.where(tk == j, lmap_ref[j], lk)
        loc.append(lk)

    oh = [[None] * _E for _ in range(_K)]
    rank = [[None] * _E for _ in range(_K)]
    c_col = [[None] * _E for _ in range(_K)]
    for k in range(_K):
        for e in range(_E):
            o = (loc[k] == e).astype(f32)
            r = mm(o, U128) + mm(L16, mm(o, ones128))
            oh[k][e] = o
            rank[k][e] = r
            c_col[k][e] = jnp.sum(o)

    start, blkstart, pad, P_cum = [], [], [], []
    s = jnp.float32(0.0)
    pcum = jnp.float32(0.0)
    for e in range(_E):
        ce = c_col[0][e] + c_col[1][e]
        used_rows = jnp.floor((ce + (_B - 1)) * (1.0 / _B)) * _B
        start.append(s)
        blkstart.append(s * (1.0 / _B))
        pad.append(used_rows - ce)
        P_cum.append(pcum)
        s = s + used_rows
        pcum = pcum + (used_rows - ce)
    total_used = s
    P_tail = pcum

    for k in range(_K):
        p = jnp.zeros((16, 128), f32)
        for e in range(_E):
            base = start[e] + (c_col[0][e] if k == 1 else 0.0)
            p = p + oh[k][e] * (base + rank[k][e])
        posall_ref[16 * k:16 * (k + 1)] = p.astype(jnp.int32)

    jmat = (lax.broadcasted_iota(i32, (16, 128), 0) * 128
            + lax.broadcasted_iota(i32, (16, 128), 1)).astype(f32)
    pp = jnp.zeros((16, 128), f32)
    for e in range(_E):
        lo, hi = P_cum[e], P_cum[e] + pad[e]
        m = jnp.logical_and(jmat >= lo, jmat < hi).astype(f32)
        pp = pp + m * (start[e] + (c_col[0][e] + c_col[1][e]) + (jmat - lo))
    mtail = (jmat >= P_tail).astype(f32)
    pp = pp + mtail * (total_used + (jmat - P_tail))
    posall_ref[32:48] = pp.astype(jnp.int32)

    bi = lax.broadcasted_iota(i32, (8, 128), 1).astype(f32)
    be = jnp.zeros((8, 128), f32)
    for e in range(_E):
        lo = blkstart[e]
        hi = blkstart[e + 1] if e + 1 < _E else total_used * (1.0 / _B)
        m = jnp.logical_and(bi >= lo, bi < hi).astype(f32)
        be = be + m * e
    bexp_ref[...] = be.astype(jnp.int32)


def _route(topk_km, w_km, lmap):
    return pl.pallas_call(
        _route_body,
        in_specs=[
            pl.BlockSpec((_K, 16, 128), lambda: (0, 0, 0)),
            pl.BlockSpec((_K, 16, 128), lambda: (0, 0, 0)),
            pl.BlockSpec(memory_space=pltpu.SMEM),
        ],
        out_specs=[
            pl.BlockSpec((48, 128), lambda: (0, 0)),
            pl.BlockSpec((8, 128), lambda: (0, 0)),
        ],
        out_shape=[
            jax.ShapeDtypeStruct((48, 128), jnp.int32),
            jax.ShapeDtypeStruct((8, 128), jnp.int32),
        ],
    )(topk_km, w_km, lmap)


def _make_gs():
    mesh = plsc.VectorSubcoreMesh(core_axis_name="c", subcore_axis_name="s")

    @functools.partial(
        pl.kernel, mesh=mesh,
        out_type=jax.ShapeDtypeStruct((_NPAD, _D), jnp.float32),
        scratch_types=[
            pltpu.VMEM_SHARED((_NPAD,), jnp.int32),
            pltpu.VMEM((3, 128), jnp.int32),
            pltpu.VMEM((128,), jnp.int32),
            pltpu.VMEM((_RPW,), jnp.int32),
            pltpu.VMEM((2, _GC, _D), jnp.float32),
            pltpu.SemaphoreType.DMA,
            pltpu.SemaphoreType.DMA,
        ],
    )
    def k(pos_hbm, tok_hbm, x_hbm, xs_hbm,
          spm, sidx_v, sval_v, gidx_v, rows_v, gsem, ssem):
        sid = lax.axis_index("s")
        cid = lax.axis_index("c")
        sbase = sid * (_NPAD // 16)
        for c in range(3):
            off = sbase + c * 128
            pltpu.sync_copy(pos_hbm.at[pl.ds(off, 128)], sidx_v.at[c])
            pltpu.sync_copy(tok_hbm.at[pl.ds(off, 128)], sval_v)
            pltpu.sync_copy(sval_v, spm.at[sidx_v.at[c]])
        plsc.subcore_barrier()
        wid = sid * 2 + cid
        base = wid * _RPW
        pltpu.sync_copy(spm.at[pl.ds(base, _RPW)], gidx_v)
        nch = _RPW // _GC
        stores = [None] * nch
        for c in range(nch):
            b = c % 2
            if c >= 2:
                stores[c - 2].wait()
            pltpu.async_copy(
                x_hbm.at[gidx_v.at[pl.ds(c * _GC, _GC)]], rows_v.at[b],
                gsem).wait()
            stores[c] = pltpu.async_copy(
                rows_v.at[b], xs_hbm.at[pl.ds(base + c * _GC, _GC)], ssem)
        for c in range(nch - 2, nch):
            stores[c].wait()

    return k


def _ffn_body(bexp_ref, xs_ref, w1_ref, w2_ref, o_ref):
    h = jnp.maximum(
        lax.dot_general(xs_ref[...], w1_ref[...], (((1,), (0,)), ((), ())),
                        preferred_element_type=jnp.float32), 0.0)
    o_ref[...] = lax.dot_general(h, w2_ref[...], (((1,), (0,)), ((), ())),
                                 preferred_element_type=jnp.float32)


def _ffn(bexp, xs, W1, W2):
    grid_spec = pltpu.PrefetchScalarGridSpec(
        num_scalar_prefetch=1,
        grid=(_NBLK,),
        in_specs=[
            pl.BlockSpec((_B, _D), lambda b, be: (b, 0)),
            pl.BlockSpec((None, _D, _F), lambda b, be: (be[b], 0, 0)),
            pl.BlockSpec((None, _F, _D), lambda b, be: (be[b], 0, 0)),
        ],
        out_specs=pl.BlockSpec((_B, _D), lambda b, be: (b, 0)),
    )
    return pl.pallas_call(
        _ffn_body,
        grid_spec=grid_spec,
        out_shape=jax.ShapeDtypeStruct((_NPAD, _D), jnp.float32),
    )(bexp, xs, W1, W2)


def _make_combine():
    mesh = plsc.VectorSubcoreMesh(core_axis_name="c", subcore_axis_name="s")

    @functools.partial(
        pl.kernel, mesh=mesh,
        out_type=jax.ShapeDtypeStruct((_T, _D), jnp.float32),
        scratch_types=[
            pltpu.VMEM((32,), jnp.int32),
            pltpu.VMEM((32, 16), jnp.float32),
            pltpu.VMEM((32, 16), jnp.float32),
            pltpu.VMEM((32, _D), jnp.float32),
            pltpu.VMEM((32, _D), jnp.float32),
            pltpu.SemaphoreType.DMA,
        ],
    )
    def k(o_hbm, pos0_hbm, pos1_hbm, w0_hbm, w1_hbm, y_hbm,
          idx_v, w0_v, w1_v, g0_v, g1_v, sem):
        wid = lax.axis_index("s") * 2 + lax.axis_index("c")
        base = wid * _TPW
        for c in range(2):
            off = base + c * 32
            pltpu.sync_copy(pos0_hbm.at[pl.ds(off, 32)], idx_v)
            pltpu.async_copy(o_hbm.at[idx_v], g0_v, sem).wait()
            pltpu.sync_copy(pos1_hbm.at[pl.ds(off, 32)], idx_v)
            pltpu.async_copy(o_hbm.at[idx_v], g1_v, sem).wait()
            pltpu.sync_copy(w0_hbm.at[pl.ds(off, 32)], w0_v)
            pltpu.sync_copy(w1_hbm.at[pl.ds(off, 32)], w1_v)

            def wsum_row(r, carry):
                a0 = w0_v[r, :]
                a1 = w1_v[r, :]
                for v in range(_D // 16):
                    sl = pl.ds(v * 16, 16)
                    g0_v[r, sl] = a0 * g0_v[r, sl] + a1 * g1_v[r, sl]
                return carry

            lax.fori_loop(0, 32, wsum_row, 0)
            pltpu.sync_copy(g0_v, y_hbm.at[pl.ds(off, 32)])

    return k


def kernel(x, topk_indices, topk_weights, W1, W2, device_indices_map,
           local_expert_indices_map):
    topk_km = topk_indices.T.reshape(_K, 16, 128)
    w_km = topk_weights.T.reshape(_K, 16, 128)
    posall, bexp = _route(topk_km, w_km, local_expert_indices_map)
    pos_flat = posall.reshape(-1)
    bexp_vec = bexp.reshape(-1)[:_NBLK]

    tok_all = jnp.concatenate([
        jnp.arange(_T, dtype=jnp.int32),
        jnp.arange(_T, dtype=jnp.int32),
        jnp.zeros((_NPAD - _P,), jnp.int32),
    ])

    xs = _make_gs()(pos_flat, tok_all, x)
    o = _ffn(bexp_vec, xs, W1, W2)
    w0b = jnp.repeat(topk_weights[:, 0:1], 16, axis=1)
    w1b = jnp.repeat(topk_weights[:, 1:2], 16, axis=1)
    y = _make_combine()(o, pos_flat[:_T], pos_flat[_T:_P], w0b, w1b)
    return y


# speedup vs baseline: 1.5339x; 1.2781x over previous
import functools

import jax
import jax.numpy as jnp
from jax import lax
from jax.experimental import pallas as pl
from jax.experimental.pallas import tpu as pltpu
from jax.experimental.pallas import tpu_sc as plsc

_E = 8
_K = 2
_T = 2048
_D = 1024
_F = 2048
_B = 256
_P = _T * _K
_NPAD = _P + _E * _B
_NBLK = _NPAD // _B
_NW = 32
_TPW = _T // _NW


def _route_body(topk_ref, w_ref, lmap_ref, posall_ref, bexp_ref):
    f32 = jnp.float32
    i32 = jnp.int32
    ii = lax.broadcasted_iota(i32, (128, 128), 0)
    jj = lax.broadcasted_iota(i32, (128, 128), 1)
    U128 = (ii < jj).astype(f32)
    ones128 = jnp.ones((128, 128), f32)
    i16 = lax.broadcasted_iota(i32, (16, 16), 0)
    j16 = lax.broadcasted_iota(i32, (16, 16), 1)
    L16 = (j16 < i16).astype(f32)

    def mm(a, b):
        return lax.dot_general(a, b, (((1,), (0,)), ((), ())),
                               preferred_element_type=f32)

    loc = []
    for k in range(_K):
        tk = topk_ref[k]
        lk = jnp.zeros_like(tk)
        for j in range(_E):
            lk = jnp.where(tk == j, lmap_ref[j], lk)
        loc.append(lk)

    oh = [[None] * _E for _ in range(_K)]
    rank = [[None] * _E for _ in range(_K)]
    c_col = [[None] * _E for _ in range(_K)]
    for k in range(_K):
        for e in range(_E):
            o = (loc[k] == e).astype(f32)
            r = mm(o, U128) + mm(L16, mm(o, ones128))
            oh[k][e] = o
            rank[k][e] = r
            c_col[k][e] = jnp.sum(o)

    start, blkstart = [], []
    s = jnp.float32(0.0)
    for e in range(_E):
        ce = c_col[0][e] + c_col[1][e]
        used_rows = jnp.floor((ce + (_B - 1)) * (1.0 / _B)) * _B
        start.append(s)
        blkstart.append(s * (1.0 / _B))
        s = s + used_rows
    total_used = s

    for k in range(_K):
        p = jnp.zeros((16, 128), f32)
        for e in range(_E):
            base = start[e] + (c_col[0][e] if k == 1 else 0.0)
            p = p + oh[k][e] * (base + rank[k][e])
        posall_ref[16 * k:16 * (k + 1)] = p.astype(jnp.int32)

    bi = lax.broadcasted_iota(i32, (8, 128), 1).astype(f32)
    be = jnp.zeros((8, 128), f32)
    for e in range(_E):
        lo = blkstart[e]
        hi = blkstart[e + 1] if e + 1 < _E else total_used * (1.0 / _B)
        m = jnp.logical_and(bi >= lo, bi < hi).astype(f32)
        be = be + m * e
    bexp_ref[...] = be.astype(jnp.int32)


def _route(topk_km, w_km, lmap):
    return pl.pallas_call(
        _route_body,
        in_specs=[
            pl.BlockSpec((_K, 16, 128), lambda: (0, 0, 0)),
            pl.BlockSpec((_K, 16, 128), lambda: (0, 0, 0)),
            pl.BlockSpec(memory_space=pltpu.SMEM),
        ],
        out_specs=[
            pl.BlockSpec((32, 128), lambda: (0, 0)),
            pl.BlockSpec((8, 128), lambda: (0, 0)),
        ],
        out_shape=[
            jax.ShapeDtypeStruct((32, 128), jnp.int32),
            jax.ShapeDtypeStruct((8, 128), jnp.int32),
        ],
    )(topk_km, w_km, lmap)


def _ffn_body(bexp_ref, pos0_ref, pos1_ref, x_ref, w1_ref, w2_ref, o_ref):
    b = pl.program_id(0)
    bf16 = jnp.bfloat16
    ri = lax.broadcasted_iota(jnp.int32, (_B, _T), 0) + b * _B
    g = jnp.logical_or(pos0_ref[...] == ri, pos1_ref[...] == ri)
    xb = lax.dot_general(g.astype(bf16), x_ref[...],
                         (((1,), (0,)), ((), ())),
                         preferred_element_type=jnp.float32)
    h = jnp.maximum(
        lax.dot_general(xb.astype(bf16), w1_ref[...],
                        (((1,), (0,)), ((), ())),
                        preferred_element_type=jnp.float32), 0.0)
    o_ref[...] = lax.dot_general(h.astype(bf16), w2_ref[...],
                                 (((1,), (0,)), ((), ())),
                                 preferred_element_type=jnp.float32)


def _ffn(bexp, pos0, pos1, xb16, W1b, W2b):
    grid_spec = pltpu.PrefetchScalarGridSpec(
        num_scalar_prefetch=1,
        grid=(_NBLK,),
        in_specs=[
            pl.BlockSpec((1, _T), lambda b, be: (0, 0)),
            pl.BlockSpec((1, _T), lambda b, be: (0, 0)),
            pl.BlockSpec((_T, _D), lambda b, be: (0, 0)),
            pl.BlockSpec((None, _D, _F), lambda b, be: (be[b], 0, 0)),
            pl.BlockSpec((None, _F, _D), lambda b, be: (be[b], 0, 0)),
        ],
        out_specs=pl.BlockSpec((_B, _D), lambda b, be: (b, 0)),
    )
    return pl.pallas_call(
        _ffn_body,
        grid_spec=grid_spec,
        out_shape=jax.ShapeDtypeStruct((_NPAD, _D), jnp.float32),
    )(bexp, pos0, pos1, xb16, W1b, W2b)


def _make_combine():
    mesh = plsc.VectorSubcoreMesh(core_axis_name="c", subcore_axis_name="s")

    @functools.partial(
        pl.kernel, mesh=mesh,
        out_type=jax.ShapeDtypeStruct((_T, _D), jnp.float32),
        scratch_types=[
            pltpu.VMEM((32,), jnp.int32),
            pltpu.VMEM((32, 16), jnp.float32),
            pltpu.VMEM((32, 16), jnp.float32),
            pltpu.VMEM((32, _D), jnp.float32),
            pltpu.VMEM((32, _D), jnp.float32),
            pltpu.SemaphoreType.DMA,
        ],
    )
    def k(o_hbm, pos0_hbm, pos1_hbm, w0_hbm, w1_hbm, y_hbm,
          idx_v, w0_v, w1_v, g0_v, g1_v, sem):
        wid = lax.axis_index("s") * 2 + lax.axis_index("c")
        base = wid * _TPW
        for c in range(2):
            off = base + c * 32
            pltpu.sync_copy(pos0_hbm.at[pl.ds(off, 32)], idx_v)
            pltpu.async_copy(o_hbm.at[idx_v], g0_v, sem).wait()
            pltpu.sync_copy(pos1_hbm.at[pl.ds(off, 32)], idx_v)
            pltpu.async_copy(o_hbm.at[idx_v], g1_v, sem).wait()
            pltpu.sync_copy(w0_hbm.at[pl.ds(off, 32)], w0_v)
            pltpu.sync_copy(w1_hbm.at[pl.ds(off, 32)], w1_v)

            def wsum_row(r, carry):
                a0 = w0_v[r, :]
                a1 = w1_v[r, :]
                for v in range(_D // 16):
                    sl = pl.ds(v * 16, 16)
                    g0_v[r, sl] = a0 * g0_v[r, sl] + a1 * g1_v[r, sl]
                return carry

            lax.fori_loop(0, 32, wsum_row, 0)
            pltpu.sync_copy(g0_v, y_hbm.at[pl.ds(off, 32)])

    return k


def kernel(x, topk_indices, topk_weights, W1, W2, device_indices_map,
           local_expert_indices_map):
    topk_km = topk_indices.T.reshape(_K, 16, 128)
    w_km = topk_weights.T.reshape(_K, 16, 128)
    posall, bexp = _route(topk_km, w_km, local_expert_indices_map)
    pos_flat = posall.reshape(-1)
    bexp_vec = bexp.reshape(-1)[:_NBLK]

    o = _ffn(bexp_vec, pos_flat[:_T].reshape(1, _T),
             pos_flat[_T:].reshape(1, _T), x.astype(jnp.bfloat16),
             W1.astype(jnp.bfloat16), W2.astype(jnp.bfloat16))
    w0b = jnp.repeat(topk_weights[:, 0:1], 16, axis=1)
    w1b = jnp.repeat(topk_weights[:, 1:2], 16, axis=1)
    y = _make_combine()(o, pos_flat[:_T], pos_flat[_T:], w0b, w1b)
    return y


# speedup vs baseline: 1.8767x; 1.2235x over previous
import functools

import jax
import jax.numpy as jnp
from jax import lax
from jax.experimental import pallas as pl
from jax.experimental.pallas import tpu as pltpu
from jax.experimental.pallas import tpu_sc as plsc

_E = 8
_K = 2
_T = 2048
_D = 1024
_F = 2048
_B = 256
_P = _T * _K
_NPAD = _P + _E * _B
_NBLK = _NPAD // _B
_NW = 32
_TPW = _T // _NW


def _route_body(topk_ref, w_ref, lmap_ref, posall_ref, bexp_ref):
    f32 = jnp.float32
    i32 = jnp.int32
    ii = lax.broadcasted_iota(i32, (128, 128), 0)
    jj = lax.broadcasted_iota(i32, (128, 128), 1)
    U128 = (ii < jj).astype(f32)
    ones128 = jnp.ones((128, 128), f32)
    i16 = lax.broadcasted_iota(i32, (16, 16), 0)
    j16 = lax.broadcasted_iota(i32, (16, 16), 1)
    L16 = (j16 < i16).astype(f32)

    def mm(a, b):
        return lax.dot_general(a, b, (((1,), (0,)), ((), ())),
                               preferred_element_type=f32)

    loc = []
    for k in range(_K):
        tk = topk_ref[k]
        lk = jnp.zeros_like(tk)
        for j in range(_E):
            lk = jnp.where(tk == j, lmap_ref[j], lk)
        loc.append(lk)

    oh = [[None] * _E for _ in range(_K)]
    rank = [[None] * _E for _ in range(_K)]
    c_col = [[None] * _E for _ in range(_K)]
    for k in range(_K):
        for e in range(_E):
            o = (loc[k] == e).astype(f32)
            r = mm(o, U128) + mm(L16, mm(o, ones128))
            oh[k][e] = o
            rank[k][e] = r
            c_col[k][e] = jnp.sum(o)

    start, blkstart = [], []
    s = jnp.float32(0.0)
    for e in range(_E):
        ce = c_col[0][e] + c_col[1][e]
        used_rows = jnp.floor((ce + (_B - 1)) * (1.0 / _B)) * _B
        start.append(s)
        blkstart.append(s * (1.0 / _B))
        s = s + used_rows
    total_used = s

    for k in range(_K):
        p = jnp.zeros((16, 128), f32)
        for e in range(_E):
            base = start[e] + (c_col[0][e] if k == 1 else 0.0)
            p = p + oh[k][e] * (base + rank[k][e])
        posall_ref[16 * k:16 * (k + 1)] = p.astype(jnp.int32)

    bi = lax.broadcasted_iota(i32, (8, 128), 1).astype(f32)
    be = jnp.zeros((8, 128), f32)
    for e in range(_E):
        lo = blkstart[e]
        hi = blkstart[e + 1] if e + 1 < _E else total_used * (1.0 / _B)
        m = jnp.logical_and(bi >= lo, bi < hi).astype(f32)
        be = be + m * e
    bexp_ref[...] = be.astype(jnp.int32)


def _route(topk_km, w_km, lmap):
    return pl.pallas_call(
        _route_body,
        in_specs=[
            pl.BlockSpec((_K, 16, 128), lambda: (0, 0, 0)),
            pl.BlockSpec((_K, 16, 128), lambda: (0, 0, 0)),
            pl.BlockSpec(memory_space=pltpu.SMEM),
        ],
        out_specs=[
            pl.BlockSpec((32, 128), lambda: (0, 0)),
            pl.BlockSpec((8, 128), lambda: (0, 0)),
        ],
        out_shape=[
            jax.ShapeDtypeStruct((32, 128), jnp.int32),
            jax.ShapeDtypeStruct((8, 128), jnp.int32),
        ],
    )(topk_km, w_km, lmap)


def _ffn_body(bexp_ref, pos0_ref, pos1_ref, x_ref, w1_ref, w2_ref, o_ref):
    b = pl.program_id(0)
    bf16 = jnp.bfloat16
    ri = lax.broadcasted_iota(jnp.int32, (_B, _T), 0) + b * _B
    g = jnp.logical_or(pos0_ref[...] == ri, pos1_ref[...] == ri)
    xb = lax.dot_general(g.astype(bf16), x_ref[...],
                         (((1,), (0,)), ((), ())),
                         preferred_element_type=jnp.float32)
    h = jnp.maximum(
        lax.dot_general(xb, w1_ref[...],
                        (((1,), (0,)), ((), ())),
                        preferred_element_type=jnp.float32), 0.0)
    o_ref[...] = lax.dot_general(h, w2_ref[...],
                                 (((1,), (0,)), ((), ())),
                                 preferred_element_type=jnp.float32)


def _ffn(bexp, pos0, pos1, xb16, W1b, W2b):
    grid_spec = pltpu.PrefetchScalarGridSpec(
        num_scalar_prefetch=1,
        grid=(_NBLK,),
        in_specs=[
            pl.BlockSpec((1, _T), lambda b, be: (0, 0)),
            pl.BlockSpec((1, _T), lambda b, be: (0, 0)),
            pl.BlockSpec((_T, _D), lambda b, be: (0, 0)),
            pl.BlockSpec((None, _D, _F), lambda b, be: (be[b], 0, 0)),
            pl.BlockSpec((None, _F, _D), lambda b, be: (be[b], 0, 0)),
        ],
        out_specs=pl.BlockSpec((_B, _D), lambda b, be: (b, 0)),
    )
    return pl.pallas_call(
        _ffn_body,
        grid_spec=grid_spec,
        out_shape=jax.ShapeDtypeStruct((_NPAD, _D), jnp.float32),
    )(bexp, pos0, pos1, xb16, W1b, W2b)


def _make_combine():
    mesh = plsc.VectorSubcoreMesh(core_axis_name="c", subcore_axis_name="s")

    @functools.partial(
        pl.kernel, mesh=mesh,
        out_type=jax.ShapeDtypeStruct((_T, _D), jnp.float32),
        scratch_types=[
            pltpu.VMEM((32,), jnp.int32),
            pltpu.VMEM((32, 16), jnp.float32),
            pltpu.VMEM((32, 16), jnp.float32),
            pltpu.VMEM((32, _D), jnp.float32),
            pltpu.VMEM((32, _D), jnp.float32),
            pltpu.SemaphoreType.DMA,
        ],
    )
    def k(o_hbm, pos0_hbm, pos1_hbm, w0_hbm, w1_hbm, y_hbm,
          idx_v, w0_v, w1_v, g0_v, g1_v, sem):
        wid = lax.axis_index("s") * 2 + lax.axis_index("c")
        base = wid * _TPW
        for c in range(2):
            off = base + c * 32
            pltpu.sync_copy(pos0_hbm.at[pl.ds(off, 32)], idx_v)
            pltpu.async_copy(o_hbm.at[idx_v], g0_v, sem).wait()
            pltpu.sync_copy(pos1_hbm.at[pl.ds(off, 32)], idx_v)
            pltpu.async_copy(o_hbm.at[idx_v], g1_v, sem).wait()
            pltpu.sync_copy(w0_hbm.at[pl.ds(off, 32)], w0_v)
            pltpu.sync_copy(w1_hbm.at[pl.ds(off, 32)], w1_v)

            def wsum_row(r, carry):
                a0 = w0_v[r, :]
                a1 = w1_v[r, :]
                for v in range(_D // 16):
                    sl = pl.ds(v * 16, 16)
                    g0_v[r, sl] = a0 * g0_v[r, sl] + a1 * g1_v[r, sl]
                return carry

            lax.fori_loop(0, 32, wsum_row, 0)
            pltpu.sync_copy(g0_v, y_hbm.at[pl.ds(off, 32)])

    return k


def kernel(x, topk_indices, topk_weights, W1, W2, device_indices_map,
           local_expert_indices_map):
    topk_km = topk_indices.T.reshape(_K, 16, 128)
    w_km = topk_weights.T.reshape(_K, 16, 128)
    posall, bexp = _route(topk_km, w_km, local_expert_indices_map)
    pos_flat = posall.reshape(-1)
    bexp_vec = bexp.reshape(-1)[:_NBLK]

    o = _ffn(bexp_vec, pos_flat[:_T].reshape(1, _T),
             pos_flat[_T:].reshape(1, _T), x.astype(jnp.bfloat16),
             W1, W2)
    w0b = jnp.repeat(topk_weights[:, 0:1], 16, axis=1)
    w1b = jnp.repeat(topk_weights[:, 1:2], 16, axis=1)
    y = _make_combine()(o, pos_flat[:_T], pos_flat[_T:], w0b, w1b)
    return y


# speedup vs baseline: 2.0092x; 1.0706x over previous
import functools

import jax
import jax.numpy as jnp
from jax import lax
from jax.experimental import pallas as pl
from jax.experimental.pallas import tpu as pltpu
from jax.experimental.pallas import tpu_sc as plsc

_E = 8
_K = 2
_T = 2048
_D = 1024
_F = 2048
_B = 256
_P = _T * _K
_NPAD = _P + _E * _B
_NBLK = _NPAD // _B
_NW = 32
_TPW = _T // _NW


def _route_body(topk_ref, lmap_ref, posall_ref, bexp_ref):
    f32 = jnp.float32
    i32 = jnp.int32
    ii = lax.broadcasted_iota(i32, (128, 128), 0)
    jj = lax.broadcasted_iota(i32, (128, 128), 1)
    U128 = (ii < jj).astype(f32)
    ones128 = jnp.ones((128, 128), f32)
    i16 = lax.broadcasted_iota(i32, (16, 16), 0)
    j16 = lax.broadcasted_iota(i32, (16, 16), 1)
    L16 = (j16 < i16).astype(f32)

    def mm(a, b):
        return lax.dot_general(a, b, (((1,), (0,)), ((), ())),
                               preferred_element_type=f32)

    loc = []
    for k in range(_K):
        tk = topk_ref[k]
        lk = jnp.zeros_like(tk)
        for j in range(_E):
            lk = jnp.where(tk == j, lmap_ref[j], lk)
        loc.append(lk)

    oh = [[None] * _E for _ in range(_K)]
    rank = [[None] * _E for _ in range(_K)]
    c_col = [[None] * _E for _ in range(_K)]
    for k in range(_K):
        for e in range(_E):
            o = (loc[k] == e).astype(f32)
            r = mm(o, U128) + mm(L16, mm(o, ones128))
            oh[k][e] = o
            rank[k][e] = r
            c_col[k][e] = jnp.sum(o)

    start, blkstart = [], []
    s = jnp.float32(0.0)
    for e in range(_E):
        ce = c_col[0][e] + c_col[1][e]
        used_rows = jnp.floor((ce + (_B - 1)) * (1.0 / _B)) * _B
        start.append(s)
        blkstart.append(s * (1.0 / _B))
        s = s + used_rows
    total_used = s

    for k in range(_K):
        p = jnp.zeros((16, 128), f32)
        for e in range(_E):
            base = start[e] + (c_col[0][e] if k == 1 else 0.0)
            p = p + oh[k][e] * (base + rank[k][e])
        posall_ref[16 * k:16 * (k + 1)] = p.astype(jnp.int32)

    bi = lax.broadcasted_iota(i32, (8, 128), 1).astype(f32)
    be = jnp.zeros((8, 128), f32)
    for e in range(_E):
        lo = blkstart[e]
        hi = blkstart[e + 1] if e + 1 < _E else total_used * (1.0 / _B)
        m = jnp.logical_and(bi >= lo, bi < hi).astype(f32)
        be = be + m * e
    be = be + (bi >= total_used * (1.0 / _B)).astype(f32) * 15.0
    bexp_ref[...] = be.astype(jnp.int32)


def _route(topk_km, lmap):
    return pl.pallas_call(
        _route_body,
        in_specs=[
            pl.BlockSpec((_K, 16, 128), lambda: (0, 0, 0)),
            pl.BlockSpec(memory_space=pltpu.SMEM),
        ],
        out_specs=[
            pl.BlockSpec((32, 128), lambda: (0, 0)),
            pl.BlockSpec((8, 128), lambda: (0, 0)),
        ],
        out_shape=[
            jax.ShapeDtypeStruct((32, 128), jnp.int32),
            jax.ShapeDtypeStruct((8, 128), jnp.int32),
        ],
    )(topk_km, lmap)


def _ffn_body(bexp_ref, pos0_ref, pos1_ref, x_ref, w1_ref, w2_ref, o_ref):
    b = pl.program_id(0)

    @pl.when(bexp_ref[b] < _E)
    def _():
        bf16 = jnp.bfloat16
        ri = lax.broadcasted_iota(jnp.int32, (_B, _T), 0) + b * _B
        g = jnp.logical_or(pos0_ref[...] == ri, pos1_ref[...] == ri)
        xb = lax.dot_general(g.astype(bf16), x_ref[...],
                             (((1,), (0,)), ((), ())),
                             preferred_element_type=jnp.float32)
        h = jnp.maximum(
            lax.dot_general(xb, w1_ref[...],
                            (((1,), (0,)), ((), ())),
                            preferred_element_type=jnp.float32), 0.0)
        o_ref[...] = lax.dot_general(h, w2_ref[...],
                                     (((1,), (0,)), ((), ())),
                                     preferred_element_type=jnp.float32)


def _ffn(bexp, pos0, pos1, xb16, W1b, W2b):
    grid_spec = pltpu.PrefetchScalarGridSpec(
        num_scalar_prefetch=1,
        grid=(_NBLK,),
        in_specs=[
            pl.BlockSpec((1, _T), lambda b, be: (0, 0)),
            pl.BlockSpec((1, _T), lambda b, be: (0, 0)),
            pl.BlockSpec((_T, _D), lambda b, be: (0, 0)),
            pl.BlockSpec((None, _D, _F), lambda b, be: (be[b] & 7, 0, 0)),
            pl.BlockSpec((None, _F, _D), lambda b, be: (be[b] & 7, 0, 0)),
        ],
        out_specs=pl.BlockSpec((_B, _D), lambda b, be: (b, 0)),
    )
    return pl.pallas_call(
        _ffn_body,
        grid_spec=grid_spec,
        out_shape=jax.ShapeDtypeStruct((_NPAD, _D), jnp.float32),
    )(bexp, pos0, pos1, xb16, W1b, W2b)


def _make_combine():
    mesh = plsc.VectorSubcoreMesh(core_axis_name="c", subcore_axis_name="s")

    @functools.partial(
        pl.kernel, mesh=mesh,
        out_type=jax.ShapeDtypeStruct((_T, _D), jnp.float32),
        scratch_types=[
            pltpu.VMEM((32,), jnp.int32),
            pltpu.VMEM((32, 16), jnp.float32),
            pltpu.VMEM((32, 16), jnp.float32),
            pltpu.VMEM((32, _D), jnp.float32),
            pltpu.VMEM((32, _D), jnp.float32),
            pltpu.SemaphoreType.DMA,
        ],
    )
    def k(o_hbm, pos0_hbm, pos1_hbm, w0_hbm, w1_hbm, y_hbm,
          idx_v, w0_v, w1_v, g0_v, g1_v, sem):
        wid = lax.axis_index("s") * 2 + lax.axis_index("c")
        base = wid * _TPW
        for c in range(2):
            off = base + c * 32
            pltpu.sync_copy(pos0_hbm.at[pl.ds(off, 32)], idx_v)
            pltpu.async_copy(o_hbm.at[idx_v], g0_v, sem).wait()
            pltpu.sync_copy(pos1_hbm.at[pl.ds(off, 32)], idx_v)
            pltpu.async_copy(o_hbm.at[idx_v], g1_v, sem).wait()
            pltpu.sync_copy(w0_hbm.at[pl.ds(off, 32)], w0_v)
            pltpu.sync_copy(w1_hbm.at[pl.ds(off, 32)], w1_v)

            def wsum_row(r, carry):
                a0 = w0_v[r, :]
                a1 = w1_v[r, :]
                for v in range(_D // 16):
                    sl = pl.ds(v * 16, 16)
                    g0_v[r, sl] = a0 * g0_v[r, sl] + a1 * g1_v[r, sl]
                return carry

            lax.fori_loop(0, 32, wsum_row, 0)
            pltpu.sync_copy(g0_v, y_hbm.at[pl.ds(off, 32)])

    return k


def kernel(x, topk_indices, topk_weights, W1, W2, device_indices_map,
           local_expert_indices_map):
    topk_km = topk_indices.T.reshape(_K, 16, 128)
    posall, bexp = _route(topk_km, local_expert_indices_map)
    pos_flat = posall.reshape(-1)
    bexp_vec = bexp.reshape(-1)[:_NBLK]

    o = _ffn(bexp_vec, pos_flat[:_T].reshape(1, _T),
             pos_flat[_T:].reshape(1, _T), x.astype(jnp.bfloat16),
             W1, W2)
    w0b = jnp.repeat(topk_weights[:, 0:1], 16, axis=1)
    w1b = jnp.repeat(topk_weights[:, 1:2], 16, axis=1)
    y = _make_combine()(o, pos_flat[:_T], pos_flat[_T:], w0b, w1b)
    return y


# speedup vs baseline: 2.0672x; 1.0289x over previous
import functools

import jax
import jax.numpy as jnp
from jax import lax
from jax.experimental import pallas as pl
from jax.experimental.pallas import tpu as pltpu
from jax.experimental.pallas import tpu_sc as plsc

_E = 8
_K = 2
_T = 2048
_D = 1024
_F = 2048
_B = 256
_P = _T * _K
_NPAD = _P + _E * _B
_NBLK = _NPAD // _B
_NW = 32
_TPW = _T // _NW


def _route_body(topk_ref, w_ref, x_ref, lmap_ref,
                posall_ref, bexp_ref, xb16_ref, w0b_ref, w1b_ref):
    xb16_ref[...] = x_ref[...].astype(jnp.bfloat16)
    w0b_ref[...] = jnp.broadcast_to(w_ref[:, 0:1], (_T, 16))
    w1b_ref[...] = jnp.broadcast_to(w_ref[:, 1:2], (_T, 16))
    f32 = jnp.float32
    i32 = jnp.int32
    ii = lax.broadcasted_iota(i32, (128, 128), 0)
    jj = lax.broadcasted_iota(i32, (128, 128), 1)
    U128 = (ii < jj).astype(f32)
    ones128 = jnp.ones((128, 128), f32)
    i16 = lax.broadcasted_iota(i32, (16, 16), 0)
    j16 = lax.broadcasted_iota(i32, (16, 16), 1)
    L16 = (j16 < i16).astype(f32)

    def mm(a, b):
        return lax.dot_general(a, b, (((1,), (0,)), ((), ())),
                               preferred_element_type=f32)

    loc = []
    for k in range(_K):
        tk = topk_ref[k]
        lk = jnp.zeros_like(tk)
        for j in range(_E):
            lk = jnp.where(tk == j, lmap_ref[j], lk)
        loc.append(lk)

    oh = [[None] * _E for _ in range(_K)]
    rank = [[None] * _E for _ in range(_K)]
    c_col = [[None] * _E for _ in range(_K)]
    for k in range(_K):
        for e in range(_E):
            o = (loc[k] == e).astype(f32)
            r = mm(o, U128) + mm(L16, mm(o, ones128))
            oh[k][e] = o
            rank[k][e] = r
            c_col[k][e] = jnp.sum(o)

    start, blkstart = [], []
    s = jnp.float32(0.0)
    for e in range(_E):
        ce = c_col[0][e] + c_col[1][e]
        used_rows = jnp.floor((ce + (_B - 1)) * (1.0 / _B)) * _B
        start.append(s)
        blkstart.append(s * (1.0 / _B))
        s = s + used_rows
    total_used = s

    for k in range(_K):
        p = jnp.zeros((16, 128), f32)
        for e in range(_E):
            base = start[e] + (c_col[0][e] if k == 1 else 0.0)
            p = p + oh[k][e] * (base + rank[k][e])
        posall_ref[16 * k:16 * (k + 1)] = p.astype(jnp.int32)

    bi = lax.broadcasted_iota(i32, (8, 128), 1).astype(f32)
    be = jnp.zeros((8, 128), f32)
    for e in range(_E):
        lo = blkstart[e]
        hi = blkstart[e + 1] if e + 1 < _E else total_used * (1.0 / _B)
        m = jnp.logical_and(bi >= lo, bi < hi).astype(f32)
        be = be + m * e
    be = be + (bi >= total_used * (1.0 / _B)).astype(f32) * 15.0
    bexp_ref[...] = be.astype(jnp.int32)


def _route(topk_km, topk_weights, x, lmap):
    return pl.pallas_call(
        _route_body,
        in_specs=[
            pl.BlockSpec((_K, 16, 128), lambda: (0, 0, 0)),
            pl.BlockSpec((_T, _K), lambda: (0, 0)),
            pl.BlockSpec((_T, _D), lambda: (0, 0)),
            pl.BlockSpec(memory_space=pltpu.SMEM),
        ],
        out_specs=[
            pl.BlockSpec((32, 128), lambda: (0, 0)),
            pl.BlockSpec((8, 128), lambda: (0, 0)),
            pl.BlockSpec((_T, _D), lambda: (0, 0)),
            pl.BlockSpec((_T, 16), lambda: (0, 0)),
            pl.BlockSpec((_T, 16), lambda: (0, 0)),
        ],
        out_shape=[
            jax.ShapeDtypeStruct((32, 128), jnp.int32),
            jax.ShapeDtypeStruct((8, 128), jnp.int32),
            jax.ShapeDtypeStruct((_T, _D), jnp.bfloat16),
            jax.ShapeDtypeStruct((_T, 16), jnp.float32),
            jax.ShapeDtypeStruct((_T, 16), jnp.float32),
        ],
    )(topk_km, topk_weights, x, lmap)


def _ffn_body(bexp_ref, pos0_ref, pos1_ref, x_ref, w1_ref, w2_ref, o_ref):
    b = pl.program_id(0)

    @pl.when(bexp_ref[b] < _E)
    def _():
        bf16 = jnp.bfloat16
        ri = lax.broadcasted_iota(jnp.int32, (_B, _T), 0) + b * _B
        g = jnp.logical_or(pos0_ref[...] == ri, pos1_ref[...] == ri)
        xb = lax.dot_general(g.astype(bf16), x_ref[...],
                             (((1,), (0,)), ((), ())),
                             preferred_element_type=jnp.float32)
        h = jnp.maximum(
            lax.dot_general(xb, w1_ref[...],
                            (((1,), (0,)), ((), ())),
                            preferred_element_type=jnp.float32), 0.0)
        o_ref[...] = lax.dot_general(h, w2_ref[...],
                                     (((1,), (0,)), ((), ())),
                                     preferred_element_type=jnp.float32)


def _ffn(bexp, pos0, pos1, xb16, W1b, W2b):
    grid_spec = pltpu.PrefetchScalarGridSpec(
        num_scalar_prefetch=1,
        grid=(_NBLK,),
        in_specs=[
            pl.BlockSpec((1, _T), lambda b, be: (0, 0)),
            pl.BlockSpec((1, _T), lambda b, be: (0, 0)),
            pl.BlockSpec((_T, _D), lambda b, be: (0, 0)),
            pl.BlockSpec((None, _D, _F), lambda b, be: (be[b] & 7, 0, 0)),
            pl.BlockSpec((None, _F, _D), lambda b, be: (be[b] & 7, 0, 0)),
        ],
        out_specs=pl.BlockSpec((_B, _D), lambda b, be: (b, 0)),
    )
    return pl.pallas_call(
        _ffn_body,
        grid_spec=grid_spec,
        out_shape=jax.ShapeDtypeStruct((_NPAD, _D), jnp.float32),
    )(bexp, pos0, pos1, xb16, W1b, W2b)


def _make_combine():
    mesh = plsc.VectorSubcoreMesh(core_axis_name="c", subcore_axis_name="s")

    @functools.partial(
        pl.kernel, mesh=mesh,
        out_type=jax.ShapeDtypeStruct((_T, _D), jnp.float32),
        scratch_types=[
            pltpu.VMEM((32,), jnp.int32),
            pltpu.VMEM((32, 16), jnp.float32),
            pltpu.VMEM((32, 16), jnp.float32),
            pltpu.VMEM((32, _D), jnp.float32),
            pltpu.VMEM((32, _D), jnp.float32),
            pltpu.SemaphoreType.DMA,
        ],
    )
    def k(o_hbm, pos0_hbm, pos1_hbm, w0_hbm, w1_hbm, y_hbm,
          idx_v, w0_v, w1_v, g0_v, g1_v, sem):
        wid = lax.axis_index("s") * 2 + lax.axis_index("c")
        base = wid * _TPW
        for c in range(2):
            off = base + c * 32
            pltpu.sync_copy(pos0_hbm.at[pl.ds(off, 32)], idx_v)
            pltpu.async_copy(o_hbm.at[idx_v], g0_v, sem).wait()
            pltpu.sync_copy(pos1_hbm.at[pl.ds(off, 32)], idx_v)
            pltpu.async_copy(o_hbm.at[idx_v], g1_v, sem).wait()
            pltpu.sync_copy(w0_hbm.at[pl.ds(off, 32)], w0_v)
            pltpu.sync_copy(w1_hbm.at[pl.ds(off, 32)], w1_v)

            def wsum_row(r, carry):
                a0 = w0_v[r, :]
                a1 = w1_v[r, :]
                for v in range(_D // 16):
                    sl = pl.ds(v * 16, 16)
                    g0_v[r, sl] = a0 * g0_v[r, sl] + a1 * g1_v[r, sl]
                return carry

            lax.fori_loop(0, 32, wsum_row, 0)
            pltpu.sync_copy(g0_v, y_hbm.at[pl.ds(off, 32)])

    return k


def kernel(x, topk_indices, topk_weights, W1, W2, device_indices_map,
           local_expert_indices_map):
    topk_km = topk_indices.T.reshape(_K, 16, 128)
    posall, bexp, xb16, w0b, w1b = _route(
        topk_km, topk_weights, x, local_expert_indices_map)
    pos_flat = posall.reshape(-1)
    bexp_vec = bexp.reshape(-1)[:_NBLK]

    o = _ffn(bexp_vec, pos_flat[:_T].reshape(1, _T),
             pos_flat[_T:].reshape(1, _T), xb16, W1, W2)
    y = _make_combine()(o, pos_flat[:_T], pos_flat[_T:], w0b, w1b)
    return y
